# Initial kernel scaffold; baseline (speedup 1.0000x reference)
import jax
import jax.numpy as jnp
from jax.experimental import pallas as pl


def kernel(distance_constraints, W_embed, ln_weight, ln_bias, W_proj):
    raise NotImplementedError("write your pallas kernel here")


# speedup vs baseline: 2.2690x; 2.2690x over previous
import functools

import jax
import jax.numpy as jnp
from jax import lax
from jax.experimental import pallas as pl
from jax.experimental.pallas import tpu as pltpu
from jax.experimental.pallas import tpu_sc as plsc

C_Z = 128
N_BINS = 64
MIN_D = 0.0
MAX_D = 50.0
N = 512
NTOT = N * N

BIN_W = jnp.float32(MAX_D / N_BINS)
INV_W = jnp.float32(N_BINS / MAX_D)
CLIP_HI = jnp.float32(MAX_D - 1e-6)

NC, NS = 2, 16
NW = NC * NS
ROWS_PER_TILE = NTOT // NW
CHUNK = 128
NCHUNK = ROWS_PER_TILE // CHUNK


def _table_body(we_ref, lnw_ref, lnb_ref, wp_ref, out_ref):
    we = we_ref[...]
    mu = jnp.mean(we, axis=1, keepdims=True)
    var = jnp.mean((we - mu) ** 2, axis=1, keepdims=True)
    x = (we - mu) / jnp.sqrt(var + 1e-5) * lnw_ref[...] + lnb_ref[...]
    out_ref[...] = lax.dot_general(x, wp_ref[...], (((1,), (1,)), ((), ())),
                                   preferred_element_type=jnp.float32)


_table_call = pl.pallas_call(
    _table_body, out_shape=jax.ShapeDtypeStruct((N_BINS, C_Z), jnp.float32))


def _bin16(d):
    d = jnp.minimum(jnp.maximum(d, jnp.float32(MIN_D)), CLIP_HI)
    c0 = jnp.clip((d * INV_W).astype(jnp.int32), 0, N_BINS - 1)
    e0 = c0.astype(jnp.float32) * BIN_W
    e1 = (c0 + 1).astype(jnp.float32) * BIN_W
    k = jnp.where(d <= e0, c0 - 1, jnp.where(d > e1, c0 + 1, c0))
    return jnp.clip(k, 0, N_BINS - 1)


@functools.cache
def _make_sc_gather():
    @functools.partial(
        pl.kernel,
        mesh=plsc.VectorSubcoreMesh(core_axis_name="c", subcore_axis_name="s"),
        out_type=jax.ShapeDtypeStruct((NTOT, C_Z), jnp.float32),
        scratch_types=[
            pltpu.VMEM((ROWS_PER_TILE,), jnp.float32),
            pltpu.VMEM((ROWS_PER_TILE,), jnp.int32),
            pltpu.VMEM((CHUNK, C_Z), jnp.float32),
            pltpu.VMEM((CHUNK, C_Z), jnp.float32),
            pltpu.SemaphoreType.DMA,
            pltpu.SemaphoreType.DMA,
        ],
    )
    def _sc_gather(d_hbm, table_hbm, out_hbm, d_v, idx_v, stage0, stage1, g0, g1):
        wid = lax.axis_index("s") * NC + lax.axis_index("c")
        base = wid * ROWS_PER_TILE
        pltpu.sync_copy(d_hbm.at[pl.ds(base, ROWS_PER_TILE)], d_v)

        def idx_body(i, carry):
            off = i * 16
            idx_v[pl.ds(off, 16)] = _bin16(d_v[pl.ds(off, 16)])
            return carry

        lax.fori_loop(0, ROWS_PER_TILE // 16, idx_body, 0)

        def chunk_body(j, carry):
            idx_slice = idx_v.at[pl.ds(j * CHUNK, CHUNK)]
            pltpu.async_copy(table_hbm.at[idx_slice], stage0, g0).wait()
            pltpu.sync_copy(stage0, out_hbm.at[pl.ds(base + j * CHUNK, CHUNK)])
            return carry

        lax.fori_loop(0, NCHUNK, chunk_body, 0)

    return _sc_gather


def kernel(distance_constraints, W_embed, ln_weight, ln_bias, W_proj):
    table = _table_call(W_embed.T, ln_weight.reshape(1, C_Z),
                        ln_bias.reshape(1, C_Z), W_proj)
    d_flat = distance_constraints.reshape(NTOT)
    out = _make_sc_gather()(d_flat, table)
    return out.reshape(1, N, N, C_Z)


# speedup vs baseline: 2.3321x; 1.0278x over previous
import functools

import jax
import jax.numpy as jnp
from jax import lax
from jax.experimental import pallas as pl
from jax.experimental.pallas import tpu as pltpu
from jax.experimental.pallas import tpu_sc as plsc

C_Z = 128
N_BINS = 64
MIN_D = 0.0
MAX_D = 50.0
N = 512
NTOT = N * N

BIN_W = jnp.float32(MAX_D / N_BINS)
INV_W = jnp.float32(N_BINS / MAX_D)
CLIP_HI = jnp.float32(MAX_D - 1e-6)

NC, NS = 2, 16
NW = NC * NS
ROWS_PER_TILE = NTOT // NW
CHUNK = 128
NCHUNK = ROWS_PER_TILE // CHUNK


def _table_body(we_ref, lnw_ref, lnb_ref, wp_ref, out_ref):
    we = we_ref[...]
    mu = jnp.mean(we, axis=1, keepdims=True)
    var = jnp.mean((we - mu) ** 2, axis=1, keepdims=True)
    x = (we - mu) / jnp.sqrt(var + 1e-5) * lnw_ref[...] + lnb_ref[...]
    out_ref[...] = lax.dot_general(x, wp_ref[...], (((1,), (1,)), ((), ())),
                                   preferred_element_type=jnp.float32)


_table_call = pl.pallas_call(
    _table_body, out_shape=jax.ShapeDtypeStruct((N_BINS, C_Z), jnp.float32))


def _bin16(d):
    d = jnp.minimum(jnp.maximum(d, jnp.float32(MIN_D)), CLIP_HI)
    c0 = jnp.clip((d * INV_W).astype(jnp.int32), 0, N_BINS - 1)
    e0 = c0.astype(jnp.float32) * BIN_W
    e1 = (c0 + 1).astype(jnp.float32) * BIN_W
    k = jnp.where(d <= e0, c0 - 1, jnp.where(d > e1, c0 + 1, c0))
    return jnp.clip(k, 0, N_BINS - 1)


SLOTS = 4


@functools.cache
def _make_sc_gather():
    scratch = [
        pltpu.VMEM((ROWS_PER_TILE,), jnp.float32),
        pltpu.VMEM((ROWS_PER_TILE,), jnp.int32),
    ]
    scratch += [pltpu.VMEM((CHUNK, C_Z), jnp.float32) for _ in range(SLOTS)]
    scratch += [pltpu.SemaphoreType.DMA for _ in range(2 * SLOTS)]

    @functools.partial(
        pl.kernel,
        mesh=plsc.VectorSubcoreMesh(core_axis_name="c", subcore_axis_name="s"),
        out_type=jax.ShapeDtypeStruct((NTOT, C_Z), jnp.float32),
        scratch_types=scratch,
    )
    def _sc_gather(d_hbm, table_hbm, out_hbm, d_v, idx_v, *bufs):
        stages = bufs[:SLOTS]
        gsems = bufs[SLOTS:2 * SLOTS]
        wsems = bufs[2 * SLOTS:]
        wid = lax.axis_index("s") * NC + lax.axis_index("c")
        base = wid * ROWS_PER_TILE
        pltpu.sync_copy(d_hbm.at[pl.ds(base, ROWS_PER_TILE)], d_v)

        def idx_body(i, carry):
            off = i * 16
            idx_v[pl.ds(off, 16)] = _bin16(d_v[pl.ds(off, 16)])
            return carry

        lax.fori_loop(0, ROWS_PER_TILE // 16, idx_body, 0)

        def g_copy(j, b):
            idx_slice = idx_v.at[pl.ds(j * CHUNK, CHUNK)]
            return pltpu.make_async_copy(table_hbm.at[idx_slice], stages[b],
                                         gsems[b])

        def w_copy(j, b):
            dst = out_hbm.at[pl.ds(base + j * CHUNK, CHUNK)]
            return pltpu.make_async_copy(stages[b], dst, wsems[b])

        for b in range(SLOTS):
            g_copy(b, b).start()

        def chunk_body(t, carry):
            j = t * SLOTS
            for b in range(SLOTS):
                g_copy(j + b, b).wait()
                w_copy(j + b, b).start()
            for b in range(SLOTS):
                w_copy(j + b, b).wait()

                @pl.when(j + b + SLOTS < NCHUNK)
                def _():
                    g_copy(j + b + SLOTS, b).start()

            return carry

        lax.fori_loop(0, NCHUNK // SLOTS, chunk_body, 0)

    return _sc_gather


def kernel(distance_constraints, W_embed, ln_weight, ln_bias, W_proj):
    table = _table_call(W_embed.T, ln_weight.reshape(1, C_Z),
                        ln_bias.reshape(1, C_Z), W_proj)
    d_flat = distance_constraints.reshape(NTOT)
    out = _make_sc_gather()(d_flat, table)
    return out.reshape(1, N, N, C_Z)
